# baseline (speedup 1.0000x reference)
import functools

import jax
import jax.numpy as jnp
from jax import lax
from jax.experimental import pallas as pl
from jax.experimental.pallas import tpu as pltpu
from jax.experimental.pallas import tpu_sc as plsc

TOPK = 2
AUX_COEF = 0.01
BMG = 256


def _router_body(x_ref, rw_ref, tri_ref,
                 p0_ref, p1_ref, p2_ref, i0_ref, i1_ref, r0_ref, r1_ref,
                 starts_ref, aux_ref,
                 cnt1_ref, psum_ref, cnt2_ref, *, nblk, ntok):
    m = pl.program_id(0)
    x = x_ref[...]
    rw = rw_ref[...]
    logits = lax.dot_general(
        x, rw, (((1,), (1,)), ((), ())), preferred_element_type=jnp.float32)
    z = logits - jnp.max(logits, axis=1, keepdims=True)
    ez = jnp.exp(z)
    p = ez / jnp.sum(ez, axis=1, keepdims=True)

    e_iota = lax.broadcasted_iota(jnp.int32, p.shape, 1)
    i0 = jnp.argmax(p, axis=1)
    oh0 = (e_iota == i0[:, None]).astype(jnp.float32)
    p0 = jnp.max(p, axis=1, keepdims=True)
    pm = jnp.where(oh0 > 0, -1.0, p)
    i1 = jnp.argmax(pm, axis=1)
    oh1 = (e_iota == i1[:, None]).astype(jnp.float32)
    p1 = jnp.max(pm, axis=1, keepdims=True)

    p0_ref[...] = p0
    p1_ref[...] = p1
    p2_ref[...] = jnp.concatenate([p0, p1], axis=1)
    i0_ref[...] = i0[:, None]
    i1_ref[...] = i1[:, None]

    @pl.when(m == 0)
    def _():
        cnt1_ref[...] = jnp.zeros_like(cnt1_ref)
        psum_ref[...] = jnp.zeros_like(psum_ref)
        cnt2_ref[...] = jnp.zeros_like(cnt2_ref)

    both = oh0 + oh1
    ex = lax.dot_general(
        tri_ref[...], both, (((1,), (0,)), ((), ())),
        preferred_element_type=jnp.float32) + cnt2_ref[...]
    r0 = jnp.sum(ex * oh0, axis=1, keepdims=True)
    r1 = jnp.sum(ex * oh1, axis=1, keepdims=True)
    r0_ref[...] = r0.astype(jnp.int32)
    r1_ref[...] = r1.astype(jnp.int32)

    cnt1_ref[...] += jnp.sum(oh0, axis=0, keepdims=True)
    psum_ref[...] += jnp.sum(p, axis=0, keepdims=True)
    cnt2_ref[...] += jnp.sum(both, axis=0, keepdims=True)

    @pl.when(m == nblk - 1)
    def _():
        n = jnp.float32(ntok)
        e = jnp.float32(cnt1_ref.shape[1])
        aux_ref[...] = jnp.reshape(
            AUX_COEF * e * jnp.sum((cnt1_ref[...] / n) * (psum_ref[...] / n)),
            (1, 1))
        bmg = jnp.float32(BMG)
        pc = jnp.ceil(cnt2_ref[...] / bmg) * bmg
        r_iota = lax.broadcasted_iota(jnp.int32, (pc.shape[1], pc.shape[1]), 0)
        c_iota = lax.broadcasted_iota(jnp.int32, (pc.shape[1], pc.shape[1]), 1)
        excl = jnp.sum(
            jnp.where(r_iota < c_iota, pc[0][:, None], 0.0),
            axis=0, keepdims=True)
        starts_ref[...] = excl.astype(jnp.int32)


def _router_call(x_flat, router_w):
    n, d = x_flat.shape
    ne = router_w.shape[0]
    BR = 512
    nblk_r = n // BR
    tri = (lax.broadcasted_iota(jnp.int32, (BR, BR), 0)
           > lax.broadcasted_iota(jnp.int32, (BR, BR), 1)).astype(jnp.float32)
    return pl.pallas_call(
        functools.partial(_router_body, nblk=nblk_r, ntok=n),
        grid=(nblk_r,),
        in_specs=[
            pl.BlockSpec((BR, d), lambda m: (m, 0)),
            pl.BlockSpec((ne, d), lambda m: (0, 0)),
            pl.BlockSpec((BR, BR), lambda m: (0, 0)),
        ],
        out_specs=[
            pl.BlockSpec((BR, 1), lambda m: (m, 0)),
            pl.BlockSpec((BR, 1), lambda m: (m, 0)),
            pl.BlockSpec((BR, 2), lambda m: (m, 0)),
        ] + [pl.BlockSpec((BR, 1), lambda m: (m, 0))] * 4 + [
            pl.BlockSpec((1, ne), lambda m: (0, 0)),
            pl.BlockSpec((1, 1), lambda m: (0, 0)),
        ],
        out_shape=[
            jax.ShapeDtypeStruct((n, 1), jnp.float32),
            jax.ShapeDtypeStruct((n, 1), jnp.float32),
            jax.ShapeDtypeStruct((n, 2), jnp.float32),
            jax.ShapeDtypeStruct((n, 1), jnp.int32),
            jax.ShapeDtypeStruct((n, 1), jnp.int32),
            jax.ShapeDtypeStruct((n, 1), jnp.int32),
            jax.ShapeDtypeStruct((n, 1), jnp.int32),
            jax.ShapeDtypeStruct((1, ne), jnp.int32),
            jax.ShapeDtypeStruct((1, 1), jnp.float32),
        ],
        scratch_shapes=[
            pltpu.VMEM((1, ne), jnp.float32),
            pltpu.VMEM((1, ne), jnp.float32),
            pltpu.VMEM((1, ne), jnp.float32),
        ],
    )(x_flat, router_w, tri)


def _pos_body(i0_ref, i1_ref, r0_ref, r1_ref, st_ref,
              pos0_ref, pos1_ref, pos2_ref, *, ne):
    i0 = i0_ref[...]
    i1 = i1_ref[...]
    pos0 = r0_ref[...]
    pos1 = r1_ref[...]
    for e in range(ne):
        st_e = st_ref[0, e]
        pos0 = pos0 + jnp.where(i0 == e, st_e, 0)
        pos1 = pos1 + jnp.where(i1 == e, st_e, 0)
    pos0_ref[...] = pos0
    pos1_ref[...] = pos1
    pos2_ref[...] = jnp.concatenate([pos0, pos1], axis=1)


def _pos_call(i0, i1, r0, r1, starts2d):
    n = i0.shape[0]
    ne = starts2d.shape[1]
    BR = 512
    return pl.pallas_call(
        functools.partial(_pos_body, ne=ne),
        grid=(n // BR,),
        in_specs=[pl.BlockSpec((BR, 1), lambda m: (m, 0))] * 4 + [
            pl.BlockSpec((1, ne), lambda m: (0, 0)),
        ],
        out_specs=[pl.BlockSpec((BR, 1), lambda m: (m, 0))] * 2 + [
            pl.BlockSpec((BR, 2), lambda m: (m, 0)),
        ],
        out_shape=[jax.ShapeDtypeStruct((n, 1), jnp.int32)] * 2 + [
            jax.ShapeDtypeStruct((n, 2), jnp.int32),
        ],
    )(i0, i1, r0, r1, starts2d)


def _ffn_body(eob_ref, xg_ref, w1_ref, w3_ref, w2_ref, pr_ref, y_ref):
    x = xg_ref[...].astype(jnp.bfloat16)
    a = lax.dot_general(
        x, w1_ref[0], (((1,), (1,)), ((), ())), preferred_element_type=jnp.float32)
    b = lax.dot_general(
        x, w3_ref[0], (((1,), (1,)), ((), ())), preferred_element_type=jnp.float32)
    h = (a * jax.nn.sigmoid(a) * b).astype(jnp.bfloat16)
    y = lax.dot_general(
        h, w2_ref[0], (((1,), (1,)), ((), ())), preferred_element_type=jnp.float32)
    y_ref[...] = y * pr_ref[...]


def _ffn_call(eob, xg, W1c, W3c, W2c, pr):
    g_blocks = xg.shape[0] // BMG
    d = xg.shape[1]
    hp = W1c.shape[1]
    return pl.pallas_call(
        _ffn_body,
        grid_spec=pltpu.PrefetchScalarGridSpec(
            num_scalar_prefetch=1,
            grid=(g_blocks,),
            in_specs=[
                pl.BlockSpec((BMG, d), lambda g, eob_r: (g, 0)),
                pl.BlockSpec((1, hp, d), lambda g, eob_r: (eob_r[g], 0, 0)),
                pl.BlockSpec((1, hp, d), lambda g, eob_r: (eob_r[g], 0, 0)),
                pl.BlockSpec((1, d, hp), lambda g, eob_r: (eob_r[g], 0, 0)),
                pl.BlockSpec((BMG, 1), lambda g, eob_r: (g, 0)),
            ],
            out_specs=pl.BlockSpec((BMG, d), lambda g, eob_r: (g, 0)),
        ),
        out_shape=jax.ShapeDtypeStruct((xg.shape[0], d), jnp.float32),
    )(eob, xg, W1c, W3c, W2c, pr)


def _make_pairscat(n, r_static, n_workers, chunk):
    mesh = plsc.VectorSubcoreMesh(core_axis_name="c", subcore_axis_name="s")
    t_per_w = n // n_workers
    nchunks = t_per_w // chunk

    @functools.partial(
        pl.kernel, mesh=mesh,
        out_type=jax.ShapeDtypeStruct((r_static,), jnp.int32),
        scratch_types=[
            pltpu.VMEM((t_per_w,), jnp.int32),
            pltpu.VMEM((t_per_w,), jnp.int32),
            pltpu.VMEM((t_per_w,), jnp.int32),
            pltpu.VMEM((t_per_w,), jnp.int32),
            pltpu.SemaphoreType.DMA,
        ],
    )
    def pairscat_k(pos0_hbm, pos1_hbm, ev_hbm, od_hbm, top_hbm,
                   pos0_v, pos1_v, ev_v, od_v, sp):
        wid = lax.axis_index("s") * 2 + lax.axis_index("c")
        t0 = wid * t_per_w
        pltpu.sync_copy(pos0_hbm.at[pl.ds(t0, t_per_w)], pos0_v)
        pltpu.sync_copy(pos1_hbm.at[pl.ds(t0, t_per_w)], pos1_v)
        pltpu.sync_copy(ev_hbm.at[pl.ds(t0, t_per_w)], ev_v)
        pltpu.sync_copy(od_hbm.at[pl.ds(t0, t_per_w)], od_v)
        cps = []
        for k in range(nchunks):
            sl = pl.ds(k * chunk, chunk)
            c0 = pltpu.make_async_copy(ev_v.at[sl], top_hbm.at[pos0_v[sl]], sp)
            c0.start()
            c1 = pltpu.make_async_copy(od_v.at[sl], top_hbm.at[pos1_v[sl]], sp)
            c1.start()
            cps += [c0, c1]
        for cp in cps:
            cp.wait()

    return pairscat_k


def _make_gather_xg(n, d, r_static, n_workers, chunk):
    mesh = plsc.VectorSubcoreMesh(core_axis_name="c", subcore_axis_name="s")
    s_per_w = r_static // n_workers
    nchunks = s_per_w // chunk

    @functools.partial(
        pl.kernel, mesh=mesh,
        out_type=[
            jax.ShapeDtypeStruct((r_static, d), jnp.float32),
            jax.ShapeDtypeStruct((r_static,), jnp.float32),
        ],
        scratch_types=[
            pltpu.VMEM((chunk, d), jnp.float32),
            pltpu.VMEM((chunk, d), jnp.float32),
            pltpu.VMEM((s_per_w,), jnp.int32),
            pltpu.VMEM((s_per_w,), jnp.int32),
            pltpu.VMEM((s_per_w,), jnp.float32),
            pltpu.SemaphoreType.DMA,
            pltpu.SemaphoreType.DMA,
            pltpu.SemaphoreType.DMA,
            pltpu.SemaphoreType.DMA,
            pltpu.SemaphoreType.DMA,
        ],
    )
    def gather_k(x_hbm, pf_hbm, top_hbm, xg_hbm, pr_hbm,
                 buf_a, buf_b, top_v, tok_v, pr_v, sga, sgb, swa, swb, sp):
        wid = lax.axis_index("s") * 2 + lax.axis_index("c")
        s0 = wid * s_per_w
        pltpu.sync_copy(top_hbm.at[pl.ds(s0, s_per_w)], top_v)

        def prep(k, _):
            sl = pl.ds(k * 16, 16)
            v = jnp.clip(top_v[sl], 0, 2 * n - 1)
            top_v[sl] = v
            tok_v[sl] = lax.shift_right_logical(v, 1)
            return 0

        lax.fori_loop(0, s_per_w // 16, prep, 0)

        prcps = []
        np_ = 0
        while np_ < s_per_w:
            w = min(128, s_per_w - np_)
            cp = pltpu.make_async_copy(
                pf_hbm.at[top_v.at[pl.ds(np_, w)]], pr_v.at[pl.ds(np_, w)], sp)
            cp.start()
            prcps.append(cp)
            np_ += w

        def gath(c, buf, sem):
            return pltpu.make_async_copy(
                x_hbm.at[tok_v.at[pl.ds(c * chunk, chunk)]], buf, sem)

        def wr(c, buf, sem):
            return pltpu.make_async_copy(
                buf, xg_hbm.at[pl.ds(s0 + c * chunk, chunk)], sem)

        gath(0, buf_a, sga).start()
        gath(1, buf_b, sgb).start()

        def body(ci, _):
            ca = 2 * ci
            cb = 2 * ci + 1
            gath(ca, buf_a, sga).wait()
            wr(ca, buf_a, swa).start()
            gath(cb, buf_b, sgb).wait()
            wr(cb, buf_b, swb).start()

            @pl.when(ci < nchunks // 2 - 1)
            def _():
                wr(ca, buf_a, swa).wait()
                gath(ca + 2, buf_a, sga).start()
                wr(cb, buf_b, swb).wait()
                gath(cb + 2, buf_b, sgb).start()

            return 0

        lax.fori_loop(0, nchunks // 2, body, 0)
        wr(nchunks - 2, buf_a, swa).wait()
        wr(nchunks - 1, buf_b, swb).wait()
        for cp in prcps:
            cp.wait()
        pltpu.sync_copy(pr_v, pr_hbm.at[pl.ds(s0, s_per_w)])

    return gather_k


def _make_combine(n, d, r_static, n_workers, chunk):
    mesh = plsc.VectorSubcoreMesh(core_axis_name="c", subcore_axis_name="s")
    t_per_w = n // n_workers

    nchunks = t_per_w // chunk

    @functools.partial(
        pl.kernel, mesh=mesh,
        out_type=jax.ShapeDtypeStruct((n, d), jnp.float32),
        scratch_types=[
            pltpu.VMEM((2 * chunk, d), jnp.float32),
            pltpu.VMEM((2 * chunk, d), jnp.float32),
            pltpu.VMEM((chunk, d), jnp.float32),
            pltpu.VMEM((chunk, d), jnp.float32),
            pltpu.VMEM((2 * t_per_w,), jnp.int32),
            pltpu.SemaphoreType.DMA,
            pltpu.SemaphoreType.DMA,
            pltpu.SemaphoreType.DMA,
            pltpu.SemaphoreType.DMA,
        ],
    )
    def combine_k(y_hbm, pos2_hbm, out_hbm,
                  buf_a, buf_b, ob_a, ob_b, pos_v, sga, sgb, soa, sob):
        wid = lax.axis_index("s") * 2 + lax.axis_index("c")
        t0 = wid * t_per_w
        pltpu.sync_copy(pos2_hbm.at[pl.ds(2 * t0, 2 * t_per_w)], pos_v)

        def gath(c, buf, sem):
            sl = pl.ds(c * 2 * chunk, 2 * chunk)
            return pltpu.make_async_copy(y_hbm.at[pos_v[sl]], buf, sem)

        def owrite(c, ob, sem):
            return pltpu.make_async_copy(
                ob, out_hbm.at[pl.ds(t0 + c * chunk, chunk)], sem)

        def adds(buf, ob):
            def seg(cc, _):
                cs = pl.ds(cc * 16, 16)
                for j in range(chunk):
                    ob[j, cs] = buf[2 * j, cs] + buf[2 * j + 1, cs]
                return 0

            lax.fori_loop(0, d // 16, seg, 0)

        gath(0, buf_a, sga).start()
        gath(1, buf_b, sgb).start()

        def body(ci, _):
            ca = 2 * ci
            cb = 2 * ci + 1
            gath(ca, buf_a, sga).wait()
            adds(buf_a, ob_a)

            @pl.when(ci < nchunks // 2 - 1)
            def _():
                gath(ca + 2, buf_a, sga).start()

            @pl.when(ci > 0)
            def _():
                owrite(ca - 2, ob_a, soa).wait()

            owrite(ca, ob_a, soa).start()

            gath(cb, buf_b, sgb).wait()
            adds(buf_b, ob_b)

            @pl.when(ci < nchunks // 2 - 1)
            def _():
                gath(cb + 2, buf_b, sgb).start()

            @pl.when(ci > 0)
            def _():
                owrite(cb - 2, ob_b, sob).wait()

            owrite(cb, ob_b, sob).start()
            return 0

        lax.fori_loop(0, nchunks // 2, body, 0)
        owrite(nchunks - 2, ob_a, soa).wait()
        owrite(nchunks - 1, ob_b, sob).wait()

    return combine_k


def kernel(x, router_w, W1, W2, W3):
    b, s, d = x.shape
    ne, hid, _ = W1.shape
    n = b * s
    x_flat = x.reshape(n, d)

    p0, p1, p2, i0, i1, r0, r1, starts2d, aux = _router_call(x_flat, router_w)
    pos0, pos1, pos2 = _pos_call(i0, i1, r0, r1, starts2d)

    r_static = TOPK * n + ne * BMG
    g_blocks = r_static // BMG

    ev = jnp.arange(0, 2 * n, 2, dtype=jnp.int32)
    ps = _make_pairscat(n, r_static, 32, 16)
    top = ps(pos0[:, 0], pos1[:, 0], ev, ev + 1)

    gx = _make_gather_xg(n, d, r_static, 32, 24)
    xg, pr = gx(x_flat, p2.reshape(-1), top)

    W1c = W1.astype(jnp.bfloat16)
    W3c = W3.astype(jnp.bfloat16)
    W2c = W2.astype(jnp.bfloat16)

    blk_iota = jnp.arange(g_blocks, dtype=jnp.int32) * BMG
    eob = (jnp.sum((starts2d[0][None, :] <= blk_iota[:, None]).astype(jnp.int32),
                   axis=1) - 1).astype(jnp.int32)

    y = _ffn_call(eob, xg, W1c, W3c, W2c, pr[:, None])

    comb = _make_combine(n, d, r_static, 32, 8)
    out = comb(y, pos2.reshape(-1))

    return out.reshape(b, s, d), aux[0, 0]


# speedup vs baseline: 1.1897x; 1.1897x over previous
import functools

import jax
import jax.numpy as jnp
from jax import lax
from jax.experimental import pallas as pl
from jax.experimental.pallas import tpu as pltpu
from jax.experimental.pallas import tpu_sc as plsc

TOPK = 2
AUX_COEF = 0.01
BMG = 256


def _router_body(x_ref, rw_ref, tri_ref,
                 p0_ref, p1_ref, i0_ref, i1_ref, r0_ref, r1_ref,
                 starts_ref, aux_ref,
                 cnt1_ref, psum_ref, cnt2_ref, *, nblk, ntok):
    m = pl.program_id(0)
    x = x_ref[...]
    rw = rw_ref[...]
    logits = lax.dot_general(
        x, rw, (((1,), (1,)), ((), ())), preferred_element_type=jnp.float32)
    z = logits - jnp.max(logits, axis=1, keepdims=True)
    ez = jnp.exp(z)
    p = ez / jnp.sum(ez, axis=1, keepdims=True)

    e_iota = lax.broadcasted_iota(jnp.int32, p.shape, 1)
    i0 = jnp.argmax(p, axis=1)
    oh0 = (e_iota == i0[:, None]).astype(jnp.float32)
    p0 = jnp.max(p, axis=1, keepdims=True)
    pm = jnp.where(oh0 > 0, -1.0, p)
    i1 = jnp.argmax(pm, axis=1)
    oh1 = (e_iota == i1[:, None]).astype(jnp.float32)
    p1 = jnp.max(pm, axis=1, keepdims=True)

    p0_ref[...] = p0
    p1_ref[...] = p1
    i0_ref[...] = i0[:, None]
    i1_ref[...] = i1[:, None]

    @pl.when(m == 0)
    def _():
        cnt1_ref[...] = jnp.zeros_like(cnt1_ref)
        psum_ref[...] = jnp.zeros_like(psum_ref)
        cnt2_ref[...] = jnp.zeros_like(cnt2_ref)

    both = oh0 + oh1
    ex = lax.dot_general(
        tri_ref[...], both, (((1,), (0,)), ((), ())),
        preferred_element_type=jnp.float32) + cnt2_ref[...]
    r0 = jnp.sum(ex * oh0, axis=1, keepdims=True)
    r1 = jnp.sum(ex * oh1, axis=1, keepdims=True)
    r0_ref[...] = r0.astype(jnp.int32)
    r1_ref[...] = r1.astype(jnp.int32)

    cnt1_ref[...] += jnp.sum(oh0, axis=0, keepdims=True)
    psum_ref[...] += jnp.sum(p, axis=0, keepdims=True)
    cnt2_ref[...] += jnp.sum(both, axis=0, keepdims=True)

    @pl.when(m == nblk - 1)
    def _():
        n = jnp.float32(ntok)
        e = jnp.float32(cnt1_ref.shape[1])
        aux_ref[...] = jnp.reshape(
            AUX_COEF * e * jnp.sum((cnt1_ref[...] / n) * (psum_ref[...] / n)),
            (1, 1))
        bmg = jnp.float32(BMG)
        pc = jnp.ceil(cnt2_ref[...] / bmg) * bmg
        r_iota = lax.broadcasted_iota(jnp.int32, (pc.shape[1], pc.shape[1]), 0)
        c_iota = lax.broadcasted_iota(jnp.int32, (pc.shape[1], pc.shape[1]), 1)
        excl = jnp.sum(
            jnp.where(r_iota < c_iota, pc[0][:, None], 0.0),
            axis=0, keepdims=True)
        starts_ref[...] = excl.astype(jnp.int32)


def _router_call(x_flat, router_w):
    n, d = x_flat.shape
    ne = router_w.shape[0]
    BR = 512
    nblk_r = n // BR
    tri = (lax.broadcasted_iota(jnp.int32, (BR, BR), 0)
           > lax.broadcasted_iota(jnp.int32, (BR, BR), 1)).astype(jnp.float32)
    return pl.pallas_call(
        functools.partial(_router_body, nblk=nblk_r, ntok=n),
        grid=(nblk_r,),
        in_specs=[
            pl.BlockSpec((BR, d), lambda m: (m, 0)),
            pl.BlockSpec((ne, d), lambda m: (0, 0)),
            pl.BlockSpec((BR, BR), lambda m: (0, 0)),
        ],
        out_specs=[pl.BlockSpec((BR, 1), lambda m: (m, 0))] * 6 + [
            pl.BlockSpec((1, ne), lambda m: (0, 0)),
            pl.BlockSpec((1, 1), lambda m: (0, 0)),
        ],
        out_shape=[
            jax.ShapeDtypeStruct((n, 1), jnp.float32),
            jax.ShapeDtypeStruct((n, 1), jnp.float32),
            jax.ShapeDtypeStruct((n, 1), jnp.int32),
            jax.ShapeDtypeStruct((n, 1), jnp.int32),
            jax.ShapeDtypeStruct((n, 1), jnp.int32),
            jax.ShapeDtypeStruct((n, 1), jnp.int32),
            jax.ShapeDtypeStruct((1, ne), jnp.int32),
            jax.ShapeDtypeStruct((1, 1), jnp.float32),
        ],
        scratch_shapes=[
            pltpu.VMEM((1, ne), jnp.float32),
            pltpu.VMEM((1, ne), jnp.float32),
            pltpu.VMEM((1, ne), jnp.float32),
        ],
    )(x_flat, router_w, tri)


def _pos_body(i0_ref, i1_ref, r0_ref, r1_ref, st_ref,
              pos0_ref, pos1_ref, pos2_ref, *, ne):
    i0 = i0_ref[...]
    i1 = i1_ref[...]
    pos0 = r0_ref[...]
    pos1 = r1_ref[...]
    for e in range(ne):
        st_e = st_ref[0, e]
        pos0 = pos0 + jnp.where(i0 == e, st_e, 0)
        pos1 = pos1 + jnp.where(i1 == e, st_e, 0)
    pos0_ref[...] = pos0
    pos1_ref[...] = pos1
    pos2_ref[...] = jnp.concatenate([pos0, pos1], axis=1)


def _pos_call(i0, i1, r0, r1, starts2d):
    n = i0.shape[0]
    ne = starts2d.shape[1]
    BR = 512
    return pl.pallas_call(
        functools.partial(_pos_body, ne=ne),
        grid=(n // BR,),
        in_specs=[pl.BlockSpec((BR, 1), lambda m: (m, 0))] * 4 + [
            pl.BlockSpec((1, ne), lambda m: (0, 0)),
        ],
        out_specs=[pl.BlockSpec((BR, 1), lambda m: (m, 0))] * 2 + [
            pl.BlockSpec((BR, 2), lambda m: (m, 0)),
        ],
        out_shape=[jax.ShapeDtypeStruct((n, 1), jnp.int32)] * 2 + [
            jax.ShapeDtypeStruct((n, 2), jnp.int32),
        ],
    )(i0, i1, r0, r1, starts2d)


def _ffn_body(eob_ref, xg_ref, w1_ref, w3_ref, w2_ref, pr_ref, y_ref):
    x = xg_ref[...].astype(jnp.bfloat16)
    a = lax.dot_general(
        x, w1_ref[0], (((1,), (1,)), ((), ())), preferred_element_type=jnp.float32)
    b = lax.dot_general(
        x, w3_ref[0], (((1,), (1,)), ((), ())), preferred_element_type=jnp.float32)
    h = (a * jax.nn.sigmoid(a) * b).astype(jnp.bfloat16)
    y = lax.dot_general(
        h, w2_ref[0], (((1,), (1,)), ((), ())), preferred_element_type=jnp.float32)
    y_ref[...] = y * pr_ref[...]


def _ffn_call(eob, xg, W1c, W3c, W2c, pr):
    g_blocks = xg.shape[0] // BMG
    d = xg.shape[1]
    hp = W1c.shape[1]
    return pl.pallas_call(
        _ffn_body,
        grid_spec=pltpu.PrefetchScalarGridSpec(
            num_scalar_prefetch=1,
            grid=(g_blocks,),
            in_specs=[
                pl.BlockSpec((BMG, d), lambda g, eob_r: (g, 0)),
                pl.BlockSpec((1, hp, d), lambda g, eob_r: (eob_r[g], 0, 0)),
                pl.BlockSpec((1, hp, d), lambda g, eob_r: (eob_r[g], 0, 0)),
                pl.BlockSpec((1, d, hp), lambda g, eob_r: (eob_r[g], 0, 0)),
                pl.BlockSpec((BMG, 1), lambda g, eob_r: (g, 0)),
            ],
            out_specs=pl.BlockSpec((BMG, d), lambda g, eob_r: (g, 0)),
        ),
        out_shape=jax.ShapeDtypeStruct((xg.shape[0], d), jnp.float32),
    )(eob, xg, W1c, W3c, W2c, pr)


def _make_scatter(n, d, r_static, n_workers, chunk):
    mesh = plsc.VectorSubcoreMesh(core_axis_name="c", subcore_axis_name="s")
    t_per_w = n // n_workers

    nchunks = t_per_w // chunk

    @functools.partial(
        pl.kernel, mesh=mesh,
        out_type=[
            jax.ShapeDtypeStruct((r_static, d), jnp.float32),
            jax.ShapeDtypeStruct((r_static,), jnp.float32),
        ],
        scratch_types=[
            pltpu.VMEM((chunk, d), jnp.float32),
            pltpu.VMEM((chunk, d), jnp.float32),
            pltpu.VMEM((chunk, d), jnp.float32),
            pltpu.VMEM((t_per_w,), jnp.int32),
            pltpu.VMEM((t_per_w,), jnp.int32),
            pltpu.VMEM((t_per_w,), jnp.float32),
            pltpu.VMEM((t_per_w,), jnp.float32),
            pltpu.SemaphoreType.DMA,
            pltpu.SemaphoreType.DMA,
            pltpu.SemaphoreType.DMA,
            pltpu.SemaphoreType.DMA,
            pltpu.SemaphoreType.DMA,
            pltpu.SemaphoreType.DMA,
            pltpu.SemaphoreType.DMA,
        ],
    )
    def scatter_k(x_hbm, pos0_hbm, pos1_hbm, p0_hbm, p1_hbm, xg_hbm, pr_hbm,
                  rows_a, rows_b, rows_c, pos0_v, pos1_v, p0_v, p1_v,
                  sla, slb, slc, ssa, ssb, ssc, sp):
        wid = lax.axis_index("s") * 2 + lax.axis_index("c")
        t0 = wid * t_per_w
        pltpu.sync_copy(pos0_hbm.at[pl.ds(t0, t_per_w)], pos0_v)
        pltpu.sync_copy(pos1_hbm.at[pl.ds(t0, t_per_w)], pos1_v)
        pltpu.sync_copy(p0_hbm.at[pl.ds(t0, t_per_w)], p0_v)
        pltpu.sync_copy(p1_hbm.at[pl.ds(t0, t_per_w)], p1_v)

        prcps = []
        for k in range(nchunks):
            sl = pl.ds(k * chunk, chunk)
            c0 = pltpu.make_async_copy(p0_v.at[sl], pr_hbm.at[pos0_v[sl]], sp)
            c0.start()
            c1 = pltpu.make_async_copy(p1_v.at[sl], pr_hbm.at[pos1_v[sl]], sp)
            c1.start()
            prcps += [c0, c1]

        def load(c, buf, sem):
            return pltpu.make_async_copy(
                x_hbm.at[pl.ds(t0 + c * chunk, chunk)], buf, sem)

        def scat2(c, buf, sem):
            sl = pl.ds(c * chunk, chunk)
            s0 = pltpu.make_async_copy(buf, xg_hbm.at[pos0_v[sl]], sem)
            s1 = pltpu.make_async_copy(buf, xg_hbm.at[pos1_v[sl]], sem)
            return s0, s1

        load(0, rows_a, sla).start()
        load(1, rows_b, slb).start()
        load(2, rows_c, slc).start()

        def half(c, buf, sl, ss, nxt):
            load(c, buf, sl).wait()
            s0, s1 = scat2(c, buf, ss)
            s0.start()
            s1.start()
            return s0, s1

        def body(ci, _):
            ca = 3 * ci
            sa = half(ca, rows_a, sla, ssa, None)
            sb = half(ca + 1, rows_b, slb, ssb, None)
            sa[0].wait()
            sa[1].wait()

            @pl.when(ci < nchunks // 3 - 1)
            def _():
                load(ca + 3, rows_a, sla).start()

            sc = half(ca + 2, rows_c, slc, ssc, None)
            sb[0].wait()
            sb[1].wait()

            @pl.when(ci < nchunks // 3 - 1)
            def _():
                load(ca + 4, rows_b, slb).start()

            sc[0].wait()
            sc[1].wait()

            @pl.when(ci < nchunks // 3 - 1)
            def _():
                load(ca + 5, rows_c, slc).start()

            return 0

        lax.fori_loop(0, nchunks // 3, body, 0)

        for c in range(3 * (nchunks // 3), nchunks):
            pltpu.sync_copy(x_hbm.at[pl.ds(t0 + c * chunk, chunk)], rows_a)
            s0, s1 = scat2(c, rows_a, ssa)
            s0.start()
            s1.start()
            s0.wait()
            s1.wait()
        for cp in prcps:
            cp.wait()

    return scatter_k


def _make_combine(n, d, r_static, n_workers, chunk):
    mesh = plsc.VectorSubcoreMesh(core_axis_name="c", subcore_axis_name="s")
    t_per_w = n // n_workers

    nchunks = t_per_w // chunk

    @functools.partial(
        pl.kernel, mesh=mesh,
        out_type=jax.ShapeDtypeStruct((n, d), jnp.float32),
        scratch_types=[
            pltpu.VMEM((2 * chunk, d), jnp.float32),
            pltpu.VMEM((2 * chunk, d), jnp.float32),
            pltpu.VMEM((chunk, d), jnp.float32),
            pltpu.VMEM((chunk, d), jnp.float32),
            pltpu.VMEM((2 * t_per_w,), jnp.int32),
            pltpu.SemaphoreType.DMA,
            pltpu.SemaphoreType.DMA,
            pltpu.SemaphoreType.DMA,
            pltpu.SemaphoreType.DMA,
        ],
    )
    def combine_k(y_hbm, pos2_hbm, out_hbm,
                  buf_a, buf_b, ob_a, ob_b, pos_v, sga, sgb, soa, sob):
        wid = lax.axis_index("s") * 2 + lax.axis_index("c")
        t0 = wid * t_per_w
        pltpu.sync_copy(pos2_hbm.at[pl.ds(2 * t0, 2 * t_per_w)], pos_v)

        def gath(c, buf, sem):
            sl = pl.ds(c * 2 * chunk, 2 * chunk)
            return pltpu.make_async_copy(y_hbm.at[pos_v[sl]], buf, sem)

        def owrite(c, ob, sem):
            return pltpu.make_async_copy(
                ob, out_hbm.at[pl.ds(t0 + c * chunk, chunk)], sem)

        def adds(buf, ob):
            def seg(cc, _):
                cs = pl.ds(cc * 16, 16)
                for j in range(chunk):
                    ob[j, cs] = buf[2 * j, cs] + buf[2 * j + 1, cs]
                return 0

            lax.fori_loop(0, d // 16, seg, 0)

        gath(0, buf_a, sga).start()
        gath(1, buf_b, sgb).start()

        def body(ci, _):
            ca = 2 * ci
            cb = 2 * ci + 1
            gath(ca, buf_a, sga).wait()

            @pl.when(ci > 0)
            def _():
                owrite(ca - 2, ob_a, soa).wait()

            adds(buf_a, ob_a)

            @pl.when(ci < nchunks // 2 - 1)
            def _():
                gath(ca + 2, buf_a, sga).start()

            owrite(ca, ob_a, soa).start()

            gath(cb, buf_b, sgb).wait()

            @pl.when(ci > 0)
            def _():
                owrite(cb - 2, ob_b, sob).wait()

            adds(buf_b, ob_b)

            @pl.when(ci < nchunks // 2 - 1)
            def _():
                gath(cb + 2, buf_b, sgb).start()

            owrite(cb, ob_b, sob).start()
            return 0

        lax.fori_loop(0, nchunks // 2, body, 0)
        owrite(nchunks - 2, ob_a, soa).wait()
        owrite(nchunks - 1, ob_b, sob).wait()

    return combine_k


def kernel(x, router_w, W1, W2, W3):
    b, s, d = x.shape
    ne, hid, _ = W1.shape
    n = b * s
    x_flat = x.reshape(n, d)

    p0, p1, i0, i1, r0, r1, starts2d, aux = _router_call(x_flat, router_w)
    pos0, pos1, pos2 = _pos_call(i0, i1, r0, r1, starts2d)

    r_static = TOPK * n + ne * BMG
    g_blocks = r_static // BMG

    scat = _make_scatter(n, d, r_static, 32, 16)
    xg, pr = scat(x_flat, pos0[:, 0], pos1[:, 0], p0[:, 0], p1[:, 0])

    W1c = W1.astype(jnp.bfloat16)
    W3c = W3.astype(jnp.bfloat16)
    W2c = W2.astype(jnp.bfloat16)

    blk_iota = jnp.arange(g_blocks, dtype=jnp.int32) * BMG
    eob = (jnp.sum((starts2d[0][None, :] <= blk_iota[:, None]).astype(jnp.int32),
                   axis=1) - 1).astype(jnp.int32)

    y = _ffn_call(eob, xg, W1c, W3c, W2c, pr[:, None])

    comb = _make_combine(n, d, r_static, 32, 8)
    out = comb(y, pos2.reshape(-1))

    return out.reshape(b, s, d), aux[0, 0]
